# baseline (speedup 1.0000x reference)
import functools

import jax
import jax.numpy as jnp
from jax import lax
from jax.experimental import pallas as pl
from jax.experimental.pallas import tpu as pltpu
from jax.experimental.pallas import tpu_sc as plsc

_NUM_TO_REPLACE = 64
_B, _N, _C = 32, 8192, 128
_ROWS = _B * _N
_BLOCK_ROWS = _N // 2


def _zero_row_ids():
    perm = jax.random.permutation(jax.random.key(42), _N)
    i_to_zero = perm[:_NUM_TO_REPLACE].astype(jnp.int32)
    rows = jnp.arange(_B, dtype=jnp.int32)[:, None] * _N + i_to_zero[None, :]
    return rows.reshape(-1)


def _copy_body(pts_ref, out_ref):
    out_ref[...] = pts_ref[...]


def _tc_copy(flat):
    return pl.pallas_call(
        _copy_body,
        grid=(_ROWS // _BLOCK_ROWS,),
        in_specs=[pl.BlockSpec((_BLOCK_ROWS, _C), lambda i: (i, 0))],
        out_specs=pl.BlockSpec((_BLOCK_ROWS, _C), lambda i: (i, 0)),
        out_shape=jax.ShapeDtypeStruct((_ROWS, _C), jnp.float32),
    )(flat)


def _sc_body(idx_hbm, out_hbm, idx_v, zeros_v, isem, zsem):
    nc = 2
    wid = lax.axis_index("s") * nc + lax.axis_index("c")
    idx_cp = pltpu.make_async_copy(
        idx_hbm.at[pl.ds(wid * _NUM_TO_REPLACE, _NUM_TO_REPLACE)], idx_v, isem)
    idx_cp.start()
    zvec = jnp.zeros((16,), jnp.float32)
    for r in range(_NUM_TO_REPLACE):
        for c in range(_C // 16):
            zeros_v[r, pl.ds(c * 16, 16)] = zvec
    idx_cp.wait()
    pltpu.async_copy(zeros_v, out_hbm.at[idx_v], zsem).wait()


_sc_scatter = functools.partial(
    pl.kernel,
    out_type=(),
    mesh=plsc.VectorSubcoreMesh(core_axis_name="c", subcore_axis_name="s"),
    scratch_types=[
        pltpu.VMEM((_NUM_TO_REPLACE,), jnp.int32),
        pltpu.VMEM((_NUM_TO_REPLACE, _C), jnp.float32),
        pltpu.SemaphoreType.DMA,
        pltpu.SemaphoreType.DMA,
    ],
)(_sc_body)


def kernel(pts):
    flat = pts.reshape(_ROWS, _C)
    idx = _zero_row_ids()
    out_ref = jax.new_ref(_tc_copy(flat))
    _sc_scatter(idx, out_ref)
    return out_ref[...].reshape(_B, _N, _C)


# speedup vs baseline: 1.0501x; 1.0501x over previous
import functools

import jax
import jax.numpy as jnp
from jax import lax
from jax.experimental import pallas as pl
from jax.experimental.pallas import tpu as pltpu
from jax.experimental.pallas import tpu_sc as plsc

_NUM_TO_REPLACE = 64
_B, _N, _C = 32, 8192, 128
_ROWS = _B * _N
_BLOCK_ROWS = _N


def _zero_row_ids():
    perm = jax.random.permutation(jax.random.key(42), _N)
    i_to_zero = perm[:_NUM_TO_REPLACE].astype(jnp.int32)
    rows = jnp.arange(_B, dtype=jnp.int32)[:, None] * _N + i_to_zero[None, :]
    return rows.reshape(-1)


def _copy_body(pts_ref, out_ref):
    out_ref[...] = pts_ref[...]


def _tc_copy(flat):
    return pl.pallas_call(
        _copy_body,
        grid=(_ROWS // _BLOCK_ROWS,),
        in_specs=[pl.BlockSpec((_BLOCK_ROWS, _C), lambda i: (i, 0))],
        out_specs=pl.BlockSpec((_BLOCK_ROWS, _C), lambda i: (i, 0)),
        out_shape=jax.ShapeDtypeStruct((_ROWS, _C), jnp.float32),
    )(flat)


def _sc_body(idx_hbm, out_hbm, idx_v, zeros_v, isem, zsem):
    nc = 2
    wid = lax.axis_index("s") * nc + lax.axis_index("c")
    idx_cp = pltpu.make_async_copy(
        idx_hbm.at[pl.ds(wid * _NUM_TO_REPLACE, _NUM_TO_REPLACE)], idx_v, isem)
    idx_cp.start()
    zvec = jnp.zeros((16,), jnp.float32)
    for r in range(_NUM_TO_REPLACE):
        for c in range(_C // 16):
            zeros_v[r, pl.ds(c * 16, 16)] = zvec
    idx_cp.wait()
    pltpu.async_copy(zeros_v, out_hbm.at[idx_v], zsem).wait()


_sc_scatter = functools.partial(
    pl.kernel,
    out_type=(),
    mesh=plsc.VectorSubcoreMesh(core_axis_name="c", subcore_axis_name="s"),
    compiler_params=pltpu.CompilerParams(
        disable_bounds_checks=True, disable_semaphore_checks=True),
    scratch_types=[
        pltpu.VMEM((_NUM_TO_REPLACE,), jnp.int32),
        pltpu.VMEM((_NUM_TO_REPLACE, _C), jnp.float32),
        pltpu.SemaphoreType.DMA,
        pltpu.SemaphoreType.DMA,
    ],
)(_sc_body)


def kernel(pts):
    flat = pts.reshape(_ROWS, _C)
    idx = _zero_row_ids()
    out_ref = jax.new_ref(_tc_copy(flat))
    _sc_scatter(idx, out_ref)
    return out_ref[...].reshape(_B, _N, _C)


# speedup vs baseline: 1.0506x; 1.0005x over previous
import functools

import jax
import jax.numpy as jnp
from jax import lax
from jax.experimental import pallas as pl
from jax.experimental.pallas import tpu as pltpu
from jax.experimental.pallas import tpu_sc as plsc

_NUM_TO_REPLACE = 64
_B, _N, _C = 32, 8192, 128
_ROWS = _B * _N
_BLOCK_ROWS = _N


def _zero_row_ids():
    perm = jax.random.permutation(jax.random.key(42), _N)
    i_to_zero = perm[:_NUM_TO_REPLACE].astype(jnp.int32)
    rows = jnp.arange(_B, dtype=jnp.int32)[:, None] * _N + i_to_zero[None, :]
    return rows.reshape(-1)


def _copy_body(pts_ref, out_ref):
    out_ref[...] = pts_ref[...]


def _tc_copy(flat):
    return pl.pallas_call(
        _copy_body,
        grid=(_ROWS // _BLOCK_ROWS,),
        in_specs=[pl.BlockSpec((_BLOCK_ROWS, _C), lambda i: (i, 0))],
        out_specs=pl.BlockSpec((_BLOCK_ROWS, _C), lambda i: (i, 0)),
        out_shape=jax.ShapeDtypeStruct((_ROWS, _C), jnp.float32),
    )(flat)


def _sc_body(idx_hbm, out_hbm, idx_v, zeros_v, isem, zsem):
    nc = 2
    wid = lax.axis_index("s") * nc + lax.axis_index("c")
    idx_cp = pltpu.make_async_copy(
        idx_hbm.at[pl.ds(wid * _NUM_TO_REPLACE, _NUM_TO_REPLACE)], idx_v, isem)
    idx_cp.start()
    zvec = jnp.zeros((16,), jnp.float32)
    for r in range(_NUM_TO_REPLACE):
        for c in range(_C // 16):
            zeros_v[r, pl.ds(c * 16, 16)] = zvec
    idx_cp.wait()
    pltpu.async_copy(zeros_v, out_hbm.at[idx_v], zsem).wait()


_sc_scatter = functools.partial(
    pl.kernel,
    out_type=(),
    mesh=plsc.VectorSubcoreMesh(core_axis_name="c", subcore_axis_name="s"),
    scratch_types=[
        pltpu.VMEM((_NUM_TO_REPLACE,), jnp.int32),
        pltpu.VMEM((_NUM_TO_REPLACE, _C), jnp.float32),
        pltpu.SemaphoreType.DMA,
        pltpu.SemaphoreType.DMA,
    ],
)(_sc_body)


def kernel(pts):
    flat = pts.reshape(_ROWS, _C)
    idx = _zero_row_ids()
    out_ref = jax.new_ref(_tc_copy(flat))
    _sc_scatter(idx, out_ref)
    return out_ref[...].reshape(_B, _N, _C)
